# baseline (speedup 1.0000x reference)
import functools

import jax
import jax.numpy as jnp
from jax import lax
from jax.experimental import pallas as pl
from jax.experimental.pallas import tpu as pltpu
from jax.experimental.pallas import tpu_sc as plsc

_NW = 32


def kernel(x):
    B, Cr3, H, W, Z = x.shape
    C = Cr3 // 8
    WZ = W * Z
    SLAB = 4 * WZ
    S = B * C * 2 * H
    T = S // _NW

    x4 = x.reshape(B, Cr3 // 4, 4, H, WZ)
    mesh = plsc.VectorSubcoreMesh(core_axis_name="c", subcore_axis_name="s")

    @functools.partial(
        pl.kernel,
        mesh=mesh,
        compiler_params=pltpu.CompilerParams(needs_layout_passes=False),
        out_type=jax.ShapeDtypeStruct((B, C, H, 2 * SLAB), jnp.float32),
        scratch_types=[
            pltpu.VMEM((4, 1, WZ), jnp.float32),
            pltpu.VMEM((4, 1, WZ), jnp.float32),
            pltpu.VMEM((SLAB,), jnp.float32),
            pltpu.VMEM((SLAB,), jnp.float32),
            pltpu.SemaphoreType.DMA((2,)),
            pltpu.SemaphoreType.DMA((2,)),
        ],
    )
    def k(x_hbm, o_hbm, in_buf0, in_buf1, out_buf0, out_buf1, in_sems, out_sems):
        in_bufs = (in_buf0, in_buf1)
        out_bufs = (out_buf0, out_buf1)
        wid = lax.axis_index("c") * 16 + lax.axis_index("s")
        iota2 = 2 * lax.iota(jnp.int32, 16)

        def decode(t):
            s = t * _NW + wid
            b = s // (C * 2 * H)
            r = s % (C * 2 * H)
            c = r // (2 * H)
            hp = r % (2 * H)
            return b, c, hp // 2, hp % 2

        def start_in(t, slot):
            b, c, h, i = decode(t)
            qc = 2 * c + i
            pltpu.make_async_copy(
                x_hbm.at[b, qc, :, pl.ds(h, 1), :],
                in_bufs[slot],
                in_sems.at[slot],
            ).start()

        def wait_in(slot):
            pltpu.make_async_copy(
                x_hbm.at[0, 0, :, pl.ds(0, 1), :],
                in_bufs[slot],
                in_sems.at[slot],
            ).wait()

        def start_out(t, slot):
            b, c, h, i = decode(t)
            pltpu.make_async_copy(
                out_bufs[slot],
                o_hbm.at[b, c, h, pl.ds(i * SLAB, SLAB)],
                out_sems.at[slot],
            ).start()

        def wait_out(slot):
            pltpu.make_async_copy(
                out_bufs[slot],
                o_hbm.at[0, 0, 0, pl.ds(0, SLAB)],
                out_sems.at[slot],
            ).wait()

        start_in(0, 0)

        def body(it, carry):
            for slot in (0, 1):
                t = it * 2 + slot
                wait_in(slot)

                @pl.when(t + 1 < T)
                def _prefetch():
                    start_in(t + 1, 1 - slot)

                @pl.when(t >= 2)
                def _drain():
                    wait_out(slot)

                dst = out_bufs[slot]
                src = in_bufs[slot]
                for m in range(4):
                    j, kk = m // 2, m % 2
                    for w in range(W):
                        for half in range(2):
                            base = 128 * w + 64 * j + 32 * half + kk
                            data = src[m, 0, pl.ds(w * Z + 16 * half, 16)]
                            plsc.store_scatter(dst, [iota2 + base], data)
                start_out(t, slot)
            return carry

        lax.fori_loop(0, T // 2, body, 0)
        wait_out(0)
        wait_out(1)

    out = k(x4)
    return out.reshape(B, C, 2 * H, 2 * W, 2 * Z)
